# Initial kernel scaffold; baseline (speedup 1.0000x reference)
import jax
import jax.numpy as jnp
from jax.experimental import pallas as pl


def kernel(x, pos, edge_index, edge_attr, sse_attr, cell_ids, cell_nodes, W0, b0, Wm, bm, Wx, bx, Wc, bc, Wu, bu):
    raise NotImplementedError("write your pallas kernel here")


# speedup vs baseline: 1.0568x; 1.0568x over previous
import functools

import jax
import jax.numpy as jnp
from jax.experimental import pallas as pl
from jax.experimental.pallas import tpu as pltpu


def _dense_body(x_ref, w_ref, b_ref, o_ref, *, relu):
    a = x_ref[...].astype(jnp.bfloat16)
    w = w_ref[...].astype(jnp.bfloat16)
    acc = jnp.dot(a, w, preferred_element_type=jnp.float32)
    acc = acc + b_ref[...]
    if relu:
        acc = jnp.maximum(acc, 0.0)
    o_ref[...] = acc


def _dense(x, w, b, relu=False, block_rows=1024):
    m, k = x.shape
    k2, n = w.shape
    assert k == k2
    pad = (-m) % block_rows
    if pad:
        x = jnp.pad(x, ((0, pad), (0, 0)))
    mp = x.shape[0]
    out = pl.pallas_call(
        functools.partial(_dense_body, relu=relu),
        grid=(mp // block_rows,),
        in_specs=[
            pl.BlockSpec((block_rows, k), lambda i: (i, 0)),
            pl.BlockSpec((k, n), lambda i: (0, 0)),
            pl.BlockSpec((1, n), lambda i: (0, 0)),
        ],
        out_specs=pl.BlockSpec((block_rows, n), lambda i: (i, 0)),
        out_shape=jax.ShapeDtypeStruct((mp, n), jnp.float32),
    )(x, w, b.reshape(1, -1))
    return out[:m] if pad else out


def kernel(x, pos, edge_index, edge_attr, sse_attr, cell_ids, cell_nodes,
           W0, b0, Wm, bm, Wx, bx, Wc, bc, Wu, bu):
    N = x.shape[0]
    C = sse_attr.shape[0]
    L_ = Wm.shape[0]
    D = W0.shape[1]
    src = edge_index[0]
    dst = edge_index[1]

    H0 = _dense(x, W0, b0)
    X = pos

    ones_e = jnp.ones((src.shape[0], 1), jnp.float32)
    deg = jnp.maximum(jax.ops.segment_sum(ones_e, dst, num_segments=N), 1.0)
    ones_p = jnp.ones((cell_ids.shape[0], 1), jnp.float32)
    cell_cnt = jnp.maximum(
        jax.ops.segment_sum(ones_p, cell_ids, num_segments=C), 1.0)
    node_cell_cnt = jnp.maximum(
        jax.ops.segment_sum(ones_p, cell_nodes, num_segments=N), 1.0)

    for l in range(L_):
        A = _dense(H0, Wm[l, :D], bm[l])
        B = _dense(H0, Wm[l, D:2 * D], jnp.zeros_like(bm[l]))
        diff = X[src] - X[dst]
        dist = jnp.sum(diff * diff, axis=1, keepdims=True)
        bf = jnp.bfloat16
        We = Wm[l, 2 * D:2 * D + 2].astype(bf).astype(jnp.float32)
        wd = Wm[l, 2 * D + 2].astype(bf).astype(jnp.float32)
        ea = edge_attr.astype(bf).astype(jnp.float32)
        dist_r = dist.astype(bf).astype(jnp.float32)
        edge_term = ea @ We + dist_r * wd
        m = jax.nn.relu(A[src] + B[dst] + edge_term)
        agg1 = jax.ops.segment_sum(m, dst, num_segments=N) / deg

        cell_h = jax.ops.segment_sum(
            H0[cell_nodes], cell_ids, num_segments=C) / cell_cnt
        cell_m = _dense(jnp.concatenate([cell_h, sse_attr], axis=1),
                        Wc[l], bc[l], relu=True, block_rows=256)
        agg2 = jax.ops.segment_sum(
            cell_m[cell_ids], cell_nodes, num_segments=N) / node_cell_cnt

        H0_upd = _dense(jnp.concatenate([H0, agg1, agg2], axis=1),
                        Wu[l], bu[l], relu=True)
        xs = m @ Wx[l] + bx[l]
        X_upd = jax.ops.segment_sum(diff * xs, dst, num_segments=N) / deg
        H0 = H0 + H0_upd
        X = X + X_upd
    return (H0, X)
